# baseline (speedup 1.0000x reference)
import functools

import jax
import jax.numpy as jnp
from jax import lax
from jax.experimental import pallas as pl
from jax.experimental.pallas import tpu as pltpu
from jax.experimental.pallas import tpu_sc as plsc

_NC = 2
_NS = 16
_NW = _NC * _NS

_D = 64
_CHUNK = 128


def _sc_gather(ids_grouped, table, n_chunks):
  n_rows = _NW * n_chunks * _CHUNK

  mesh = plsc.VectorSubcoreMesh(core_axis_name="c", subcore_axis_name="s")

  @functools.partial(
      pl.kernel,
      out_type=jax.ShapeDtypeStruct((n_rows, _D), jnp.float32),
      mesh=mesh,
      scratch_types=[
          pltpu.VMEM((n_chunks, _CHUNK), jnp.int32),
          pltpu.VMEM((2, _CHUNK, _D), jnp.float32),
          pltpu.SemaphoreType.DMA,
      ],
      compiler_params=pltpu.CompilerParams(use_tc_tiling_on_sc=False),
  )
  def gather_kernel(ids_hbm, table_hbm, out_hbm, idx_v, rows_v, gsem):
    wid = lax.axis_index("s") * _NC + lax.axis_index("c")
    base = wid * (n_chunks * _CHUNK)
    pltpu.sync_copy(ids_hbm.at[wid], idx_v)

    pltpu.async_copy(table_hbm.at[idx_v.at[0]], rows_v.at[0], gsem)

    def body(j, _):
      slot = lax.rem(j, 2)
      nslot = lax.rem(j + 1, 2)

      pltpu.make_async_copy(table_hbm.at[idx_v.at[j]], rows_v.at[slot],
                            gsem).wait()

      @pl.when(j + 1 < n_chunks)
      def _():
        pltpu.async_copy(table_hbm.at[idx_v.at[j + 1]], rows_v.at[nslot],
                         gsem)

      pltpu.sync_copy(rows_v.at[slot],
                      out_hbm.at[pl.ds(base + j * _CHUNK, _CHUNK)])
      return 0

    lax.fori_loop(0, n_chunks, body, 0)

  return gather_kernel(ids_grouped, table)


def _tc_project(x, wt, b2d, blk):
  n = x.shape[0]

  def mm_kernel(x_ref, wt_ref, b_ref, o_ref):
    o_ref[...] = (
        jnp.dot(x_ref[...], wt_ref[...], preferred_element_type=jnp.float32)
        + b_ref[...])

  return pl.pallas_call(
      mm_kernel,
      out_shape=jax.ShapeDtypeStruct((n, _D), jnp.float32),
      grid=(n // blk,),
      in_specs=[
          pl.BlockSpec((blk, _D), lambda i: (i, 0)),
          pl.BlockSpec((_D, _D), lambda i: (0, 0)),
          pl.BlockSpec((1, _D), lambda i: (0, 0)),
      ],
      out_specs=pl.BlockSpec((blk, _D), lambda i: (i, 0)),
  )(x, wt, b2d)


def kernel(item_ids, feat_matrix, W, b):
  B, L = item_ids.shape
  n = B * L
  n_chunks = n // (_NW * _CHUNK)
  assert n == _NW * n_chunks * _CHUNK

  ids_grouped = item_ids.reshape(_NW, n_chunks, _CHUNK)
  rows = _sc_gather(ids_grouped, feat_matrix, n_chunks)
  out = _tc_project(rows, W.T, b.reshape(1, _D), blk=2048)
  return out.reshape(B, L, _D)
